# baseline (speedup 1.0000x reference)
import functools
import math

import jax
import jax.numpy as jnp
from jax import lax
from jax.experimental import pallas as pl
from jax.experimental.pallas import tpu as pltpu
from jax.experimental.pallas import tpu_sc as plsc

EMB = 64
PAIR = 2 * EMB
VOCAB_PAIRS = 500000
SCALE = 8.0
LANES = 16

NC = 2
NS = 16
NW = NC * NS

NBATCH = 4096
SEQ = 200
B_TOTAL = NBATCH * SEQ
ROWS_PER_W = B_TOTAL // NW
BATCH_PER_W = NBATCH // NW
GROUP = 40
SUBS = SEQ // GROUP
N_GROUPS = ROWS_PER_W // GROUP
NBUF = 3

_mesh = plsc.VectorSubcoreMesh(
    core_axis_name="c", subcore_axis_name="s", num_cores=NC, num_subcores=NS)


@functools.partial(
    pl.kernel,
    out_type=jax.ShapeDtypeStruct((NBATCH, SEQ, EMB), jnp.float32),
    mesh=_mesh,
    scratch_types=[
        pltpu.VMEM((ROWS_PER_W + LANES,), jnp.int32),
        pltpu.VMEM((ROWS_PER_W,), jnp.int32),
    ] + [pltpu.VMEM((GROUP, PAIR), jnp.float32) for _ in range(NBUF)]
      + [pltpu.VMEM((GROUP, EMB), jnp.float32) for _ in range(NBUF)]
      + [pltpu.SemaphoreType.DMA for _ in range(2 * NBUF)],
    compiler_params=pltpu.CompilerParams(use_tc_tiling_on_sc=False),
)
def _emb_lookup(tok_hbm, table_hbm, out_hbm, tok_v, idx_v,
                a0, a1, a2, b0, b1, b2,
                g0, g1, g2, o0, o1, o2):
    gbufs = [a0, a1, a2]
    obufs = [b0, b1, b2]
    gsems = [g0, g1, g2]
    osems = [o0, o1, o2]

    wid = lax.axis_index("s") * NC + lax.axis_index("c")
    pltpu.sync_copy(tok_hbm.at[pl.ds(wid * ROWS_PER_W, ROWS_PER_W)],
                    tok_v.at[pl.ds(0, ROWS_PER_W)])
    batch_base = wid * BATCH_PER_W

    def idx_body(i, carry):
        sl = pl.ds(i * LANES, LANES)
        idx_v[sl] = lax.shift_right_logical(tok_v[sl], 1)
        return carry

    lax.fori_loop(0, ROWS_PER_W // LANES, idx_body, 0, unroll=8)

    def fire_gather(k, j):
        pltpu.async_copy(
            table_hbm.at[idx_v.at[pl.ds(k * GROUP, GROUP)]],
            gbufs[j], gsems[j])

    def wait_gather(j):
        pltpu.make_async_copy(
            table_hbm.at[idx_v.at[pl.ds(0, GROUP)]],
            gbufs[j], gsems[j]).wait()

    def out_slice(k):
        return out_hbm.at[batch_base + k // SUBS].at[
            pl.ds((k % SUBS) * GROUP, GROUP)]

    def fire_out(k, j):
        pltpu.async_copy(obufs[j], out_slice(k), osems[j])

    def wait_out(j):
        pltpu.make_async_copy(obufs[j], out_slice(0), osems[j]).wait()

    def extract(k, j):
        gbuf, obuf = gbufs[j], obufs[j]
        base = k * GROUP

        def body(r, carry):
            tv = tok_v[pl.ds(base + (r // LANES) * LANES, LANES)]
            lane = jnp.full((LANES,), r % LANES, jnp.int32)
            tr = tv.at[lane].get(mode="promise_in_bounds")
            par = jnp.bitwise_and(tr, 1).astype(jnp.float32)
            for l in range(EMB // LANES):
                lo = gbuf[r, pl.ds(l * LANES, LANES)]
                hi = gbuf[r, pl.ds(EMB + l * LANES, LANES)]
                obuf[r, pl.ds(l * LANES, LANES)] = (
                    lo + par * (hi - lo)) * SCALE
            return carry

        lax.fori_loop(0, GROUP, body, 0, unroll=4)

    def step(k, j, fire_k=None, fire_slot=None, first=False):
        if not first:
            wait_out(j)
        if fire_k is not None:
            fire_gather(fire_k, fire_slot)
        wait_gather(j)
        extract(k, j)
        fire_out(k, j)

    fire_gather(0, 0)
    fire_gather(1, 1)

    step(0, 0, fire_k=2, fire_slot=2, first=True)
    step(1, 1, fire_k=3, fire_slot=0, first=True)
    step(2, 2, fire_k=4, fire_slot=1, first=True)

    def loop_body(t, carry):
        for jj in range(NBUF):
            k = NBUF * t + NBUF + jj
            step(k, jj, fire_k=k + 2, fire_slot=(jj + 2) % NBUF)
        return carry

    lax.fori_loop(0, (N_GROUPS - NBUF - 4) // NBUF, loop_body, 0)

    step(N_GROUPS - 4, (N_GROUPS - 4) % NBUF,
         fire_k=N_GROUPS - 2, fire_slot=(N_GROUPS - 2) % NBUF)
    step(N_GROUPS - 3, (N_GROUPS - 3) % NBUF,
         fire_k=N_GROUPS - 1, fire_slot=(N_GROUPS - 1) % NBUF)
    step(N_GROUPS - 2, (N_GROUPS - 2) % NBUF)
    step(N_GROUPS - 1, (N_GROUPS - 1) % NBUF)

    for k in (N_GROUPS - 3, N_GROUPS - 2, N_GROUPS - 1):
        wait_out(k % NBUF)


def kernel(tokens, embedding):
    tok = jnp.maximum(tokens.astype(jnp.int32), 0).reshape(B_TOTAL)
    return _emb_lookup(tok, embedding.reshape(VOCAB_PAIRS, PAIR))


# speedup vs baseline: 1.6378x; 1.6378x over previous
import functools
import math

import jax
import jax.numpy as jnp
from jax import lax
from jax.experimental import pallas as pl
from jax.experimental.pallas import tpu as pltpu
from jax.experimental.pallas import tpu_sc as plsc

EMB = 64
SCALE = 8.0

NC = 2
NS = 16
NW = NC * NS

NBATCH = 4096
SEQ = 200
B_TOTAL = NBATCH * SEQ
ROWS_PER_W = B_TOTAL // NW
BATCH_PER_W = NBATCH // NW
CHUNK = 128
NBUF = 4

_mesh = plsc.VectorSubcoreMesh(
    core_axis_name="c", subcore_axis_name="s", num_cores=NC, num_subcores=NS)


@functools.partial(
    pl.kernel,
    out_type=jax.ShapeDtypeStruct((NBATCH, SEQ, EMB), jnp.float32),
    mesh=_mesh,
    scratch_types=[
        pltpu.VMEM((ROWS_PER_W,), jnp.int32),
    ] + [pltpu.VMEM((SEQ, EMB), jnp.float32) for _ in range(NBUF)]
      + [pltpu.SemaphoreType.DMA for _ in range(2 * NBUF)],
    compiler_params=pltpu.CompilerParams(use_tc_tiling_on_sc=False),
)
def _emb_lookup(tok_hbm, table_hbm, out_hbm, idx_v,
                a0, a1, a2, a3,
                g0, g1, g2, g3, o0, o1, o2, o3):
    bufs = [a0, a1, a2, a3]
    gsems = [g0, g1, g2, g3]
    osems = [o0, o1, o2, o3]

    wid = lax.axis_index("s") * NC + lax.axis_index("c")
    pltpu.sync_copy(tok_hbm.at[pl.ds(wid * ROWS_PER_W, ROWS_PER_W)], idx_v)
    batch_base = wid * BATCH_PER_W

    def fire_gather(k, j):
        pltpu.async_copy(
            table_hbm.at[idx_v.at[pl.ds(k * SEQ, CHUNK)]],
            bufs[j].at[pl.ds(0, CHUNK)], gsems[j])
        pltpu.async_copy(
            table_hbm.at[idx_v.at[pl.ds(k * SEQ + CHUNK, SEQ - CHUNK)]],
            bufs[j].at[pl.ds(CHUNK, SEQ - CHUNK)], gsems[j])

    def wait_gather(j):
        pltpu.make_async_copy(
            table_hbm.at[idx_v.at[pl.ds(0, CHUNK)]],
            bufs[j].at[pl.ds(0, CHUNK)], gsems[j]).wait()
        pltpu.make_async_copy(
            table_hbm.at[idx_v.at[pl.ds(0, SEQ - CHUNK)]],
            bufs[j].at[pl.ds(CHUNK, SEQ - CHUNK)], gsems[j]).wait()

    def fire_out(k, j):
        pltpu.async_copy(bufs[j], out_hbm.at[batch_base + k], osems[j])

    def wait_out(j):
        pltpu.make_async_copy(bufs[j], out_hbm.at[0], osems[j]).wait()

    def scale(j):
        buf = bufs[j]

        def body(r, carry):
            for l in range(EMB // 16):
                sl = pl.ds(l * 16, 16)
                buf[r, sl] = buf[r, sl] * SCALE
            return carry

        lax.fori_loop(0, SEQ, body, 0, unroll=8)

    fire_gather(0, 0)
    fire_gather(1, 1)

    for k in (0, 1):
        fire_gather(k + 2, (k + 2) % NBUF)
        wait_gather(k % NBUF)
        scale(k % NBUF)
        fire_out(k, k % NBUF)

    def loop_body(t, carry):
        for jj in range(NBUF):
            k = NBUF * t + 2 + jj
            j = (2 + jj) % NBUF
            wait_out(jj)
            fire_gather(k + 2, jj)
            wait_gather(j)
            scale(j)
            fire_out(k, j)
        return carry

    lax.fori_loop(0, (BATCH_PER_W - NBUF) // NBUF, loop_body, 0)

    for k in (BATCH_PER_W - 2, BATCH_PER_W - 1):
        wait_out((k + 2) % NBUF)
        wait_gather(k % NBUF)
        scale(k % NBUF)
        fire_out(k, k % NBUF)

    wait_out((BATCH_PER_W - 2) % NBUF)
    wait_out((BATCH_PER_W - 1) % NBUF)


def kernel(tokens, embedding):
    tok = jnp.maximum(tokens.astype(jnp.int32), 0).reshape(B_TOTAL)
    return _emb_lookup(tok, embedding)


# speedup vs baseline: 1.6434x; 1.0034x over previous
import functools
import math

import jax
import jax.numpy as jnp
from jax import lax
from jax.experimental import pallas as pl
from jax.experimental.pallas import tpu as pltpu
from jax.experimental.pallas import tpu_sc as plsc

EMB = 64
SCALE = 8.0

NC = 2
NS = 16
NW = NC * NS

NBATCH = 4096
SEQ = 200
B_TOTAL = NBATCH * SEQ
ROWS_PER_W = B_TOTAL // NW
BATCH_PER_W = NBATCH // NW
CHUNK = 128
NBUF = 4

_mesh = plsc.VectorSubcoreMesh(
    core_axis_name="c", subcore_axis_name="s", num_cores=NC, num_subcores=NS)


@functools.partial(
    pl.kernel,
    out_type=jax.ShapeDtypeStruct((NBATCH, SEQ, EMB), jnp.float32),
    mesh=_mesh,
    scratch_types=[
        pltpu.VMEM((ROWS_PER_W,), jnp.int32),
    ] + [pltpu.VMEM((SEQ, EMB), jnp.float32) for _ in range(NBUF)]
      + [pltpu.SemaphoreType.DMA for _ in range(2 * NBUF)],
    compiler_params=pltpu.CompilerParams(
        use_tc_tiling_on_sc=False, allow_input_fusion=[False, True]),
)
def _emb_lookup(tok_hbm, table_hbm, out_hbm, idx_v,
                a0, a1, a2, a3,
                g0, g1, g2, g3, o0, o1, o2, o3):
    bufs = [a0, a1, a2, a3]
    gsems = [g0, g1, g2, g3]
    osems = [o0, o1, o2, o3]

    wid = lax.axis_index("s") * NC + lax.axis_index("c")
    pltpu.sync_copy(tok_hbm.at[pl.ds(wid * ROWS_PER_W, ROWS_PER_W)], idx_v)
    batch_base = wid * BATCH_PER_W

    def fire_gather(k, j):
        pltpu.async_copy(
            table_hbm.at[idx_v.at[pl.ds(k * SEQ, CHUNK)]],
            bufs[j].at[pl.ds(0, CHUNK)], gsems[j])
        pltpu.async_copy(
            table_hbm.at[idx_v.at[pl.ds(k * SEQ + CHUNK, SEQ - CHUNK)]],
            bufs[j].at[pl.ds(CHUNK, SEQ - CHUNK)], gsems[j])

    def wait_gather(j):
        pltpu.make_async_copy(
            table_hbm.at[idx_v.at[pl.ds(0, CHUNK)]],
            bufs[j].at[pl.ds(0, CHUNK)], gsems[j]).wait()
        pltpu.make_async_copy(
            table_hbm.at[idx_v.at[pl.ds(0, SEQ - CHUNK)]],
            bufs[j].at[pl.ds(CHUNK, SEQ - CHUNK)], gsems[j]).wait()

    def fire_out(k, j):
        pltpu.async_copy(bufs[j], out_hbm.at[batch_base + k], osems[j])

    def wait_out(j):
        pltpu.make_async_copy(bufs[j], out_hbm.at[0], osems[j]).wait()

    def scale(j):
        buf = bufs[j]

        def body(r, carry):
            for l in range(EMB // 16):
                sl = pl.ds(l * 16, 16)
                buf[r, sl] = buf[r, sl] * SCALE
            return carry

        lax.fori_loop(0, SEQ, body, 0, unroll=8)

    fire_gather(0, 0)
    fire_gather(1, 1)

    for k in (0, 1):
        fire_gather(k + 2, (k + 2) % NBUF)
        wait_gather(k % NBUF)
        scale(k % NBUF)
        fire_out(k, k % NBUF)

    def loop_body(t, carry):
        for jj in range(NBUF):
            k = NBUF * t + 2 + jj
            j = (2 + jj) % NBUF
            wait_out(jj)
            fire_gather(k + 2, jj)
            wait_gather(j)
            scale(j)
            fire_out(k, j)
        return carry

    lax.fori_loop(0, (BATCH_PER_W - NBUF) // NBUF, loop_body, 0)

    for k in (BATCH_PER_W - 2, BATCH_PER_W - 1):
        wait_out((k + 2) % NBUF)
        wait_gather(k % NBUF)
        scale(k % NBUF)
        fire_out(k, k % NBUF)

    wait_out((BATCH_PER_W - 2) % NBUF)
    wait_out((BATCH_PER_W - 1) % NBUF)


def kernel(tokens, embedding):
    tok = jnp.maximum(tokens.astype(jnp.int32), 0).reshape(B_TOTAL)
    return _emb_lookup(tok, embedding)
